# baseline (speedup 1.0000x reference)
import functools

import jax
import jax.numpy as jnp
from jax import lax
from jax.experimental import pallas as pl
from jax.experimental.pallas import tpu as pltpu
from jax.experimental.pallas import tpu_sc as plsc

N = 10000
E = 320000
D = 128
G = 16
OUT = 5

NC = 2
NS = 16
L = 16
NW = NC * NS

C = 120
NBUF = 3
IB = 4
STEP = 12
CH = -(-(-(-E // (NW * C))) // STEP) * STEP
E_PAD = NW * C * CH
ACC_ROWS = 10112
ZROWS = ACC_ROWS // NS
ZCH = -(-ZROWS // C)
WB = N // NS // 8 * 8
WB_LAST = N - (NS - 1) * WB


def _sc_body(x_hbm, sd_hbm, ew_hbm, w_hbm, out_hbm,
             sd_c, ew_c, rows0, rows1, rows2, w_v,
             gs0, gs1, gs2, ss0, ss1, ss2, is0, is1, is2, is3, acc):
    rows = [rows0, rows1, rows2]
    gsem = [gs0, gs1, gs2]
    ssem = [ss0, ss1, ss2]
    isem = [is0, is1, is2, is3]
    c = lax.axis_index("c")
    s = lax.axis_index("s")
    w = c * NS + s

    pltpu.sync_copy(w_hbm, w_v)

    def zrow(i, _):
        for f in range(D // L):
            rows0[i, pl.ds(f * L, L)] = jnp.zeros((L,), jnp.float32)
        return 0
    lax.fori_loop(0, C, zrow, 0)
    for k in range(ZCH):
        off = min(k * C, ZROWS - C)
        pltpu.sync_copy(rows0, acc.at[pl.ds(s * ZROWS + off, C)])
    plsc.subcore_barrier()

    wecol = [w_v[0, pl.ds(f * L, L)] for f in range(D // L)]
    bev = [w_v[1, pl.ds(f * L, L)] for f in range(D // L)]

    def idx_descs(ci, j):
        g = w * CH + ci
        return (pltpu.make_async_copy(sd_hbm.at[pl.ds(2 * g, 2)],
                                      sd_c.at[pl.ds(2 * j, 2)], isem[j]),
                pltpu.make_async_copy(ew_hbm.at[pl.ds(g * C, C)],
                                      ew_c.at[pl.ds(j * C, C)], isem[j]))

    def idx_start(ci, j):
        for d in idx_descs(ci, j):
            d.start()

    def idx_wait(ci, j):
        for d in idx_descs(ci, j):
            d.wait()

    def gather_desc(j, b):
        return pltpu.make_async_copy(
            x_hbm.at[sd_c.at[2 * j]], rows[b], gsem[b])

    def scatter_desc(j, b):
        return pltpu.make_async_copy(
            rows[b], acc.at[sd_c.at[2 * j + 1]], ssem[b])

    idx_start(0, 0)
    idx_start(1, 1)
    idx_wait(0, 0)
    gather_desc(0, 0).start()

    def outer(k, _):
        for j in range(STEP):
            ci = k * STEP + j
            b = j % NBUF
            b1 = (j + 1) % NBUF
            jj = j % IB

            @pl.when(jnp.logical_and(ci >= 2, ci + 1 < CH))
            def _():
                scatter_desc((jj + 2) % IB, b1).wait()

            @pl.when(ci + 1 < CH)
            def _():
                idx_wait(ci + 1, (jj + 1) % IB)
                gather_desc((jj + 1) % IB, b1).start()

            gather_desc(jj, b).wait()

            @pl.when(ci + 2 < CH)
            def _():
                idx_start(ci + 2, (jj + 2) % IB)

            def edge(i2, _):
                for u in range(2):
                    i = i2 * 2 + u
                    ewv = plsc.load_gather(
                        ew_c, [jnp.full((L,), jj * C + u, jnp.int32) + i2 * 2])
                    for f in range(D // L):
                        v = rows[b][i, pl.ds(f * L, L)]
                        rows[b][i, pl.ds(f * L, L)] = jnp.maximum(
                            v + ewv * wecol[f] + bev[f], 0.0)
                return 0
            lax.fori_loop(0, C // 2, edge, 0)
            scatter_desc(jj, b).start(add=True)
        return 0
    lax.fori_loop(0, CH // STEP, outer, 0)
    for t in range(NBUF):
        ci = CH - NBUF + t
        scatter_desc(ci % IB, ci % NBUF).wait()
    plsc.subcore_barrier()

    @pl.when(s < NS - 1)
    def _():
        pltpu.sync_copy(acc.at[pl.ds(s * WB, WB)],
                        out_hbm.at[c, pl.ds(s * WB, WB)])

    @pl.when(s == NS - 1)
    def _():
        pltpu.sync_copy(acc.at[pl.ds((NS - 1) * WB, WB_LAST)],
                        out_hbm.at[c, pl.ds((NS - 1) * WB, WB_LAST)])


@jax.jit
def _sc_layer(x, sd, ew2, wparams):
    mesh = plsc.VectorSubcoreMesh(core_axis_name="c", subcore_axis_name="s")
    return pl.kernel(
        _sc_body,
        out_type=jax.ShapeDtypeStruct((NC, N, D), jnp.float32),
        mesh=mesh,
        scratch_types=[
            pltpu.VMEM((IB * 2, C), jnp.int32),
            pltpu.VMEM((IB * C,), jnp.float32),
            pltpu.VMEM((C, D), jnp.float32),
            pltpu.VMEM((C, D), jnp.float32),
            pltpu.VMEM((C, D), jnp.float32),
            pltpu.VMEM((2, D), jnp.float32),
        ] + [pltpu.SemaphoreType.DMA] * (NBUF * 2 + IB) + [
            pltpu.VMEM_SHARED((ACC_ROWS, D), jnp.float32),
        ],
        compiler_params=pltpu.CompilerParams(needs_layout_passes=False),
    )(x, sd, ew2, wparams)


def _mlp_body(x_ref, a0_ref, a1_ref, w1_ref, b1_ref, g_ref, bt_ref,
              w2_ref, b2_ref, o_ref):
    h = x_ref[...] + a0_ref[...] + a1_ref[...]
    h1 = jnp.dot(h, w1_ref[...], preferred_element_type=jnp.float32) + b1_ref[...]
    mu = jnp.mean(h1, axis=0, keepdims=True)
    var = jnp.mean((h1 - mu) ** 2, axis=0, keepdims=True)
    hn = (h1 - mu) * lax.rsqrt(var + 1e-5) * g_ref[...] + bt_ref[...]
    r = jnp.maximum(hn, 0.0)
    o_ref[...] = jnp.maximum(
        jnp.dot(r, w2_ref[...], preferred_element_type=jnp.float32) + b2_ref[...],
        0.0)


@jax.jit
def _mlp_layer(x, a0, a1, w1t, b1, g, bt, w2t, b2):
    return pl.pallas_call(
        _mlp_body,
        out_shape=jax.ShapeDtypeStruct((N, D), jnp.float32),
    )(x, a0, a1, w1t, b1.reshape(1, D), g.reshape(1, D), bt.reshape(1, D),
      w2t, b2.reshape(1, D))


def _final_body(x_ref, a0_ref, a1_ref, w1_ref, b1_ref, g_ref, bt_ref,
                w2_ref, b2_ref, batch_ref, wo_ref, bo_ref, o_ref):
    h = x_ref[...] + a0_ref[...] + a1_ref[...]
    h1 = jnp.dot(h, w1_ref[...], preferred_element_type=jnp.float32) + b1_ref[...]
    mu = jnp.mean(h1, axis=0, keepdims=True)
    var = jnp.mean((h1 - mu) ** 2, axis=0, keepdims=True)
    hn = (h1 - mu) * lax.rsqrt(var + 1e-5) * g_ref[...] + bt_ref[...]
    r = jnp.maximum(hn, 0.0)
    x2 = jnp.maximum(
        jnp.dot(r, w2_ref[...], preferred_element_type=jnp.float32) + b2_ref[...],
        0.0)
    b = batch_ref[...]
    neg = jnp.float32(-jnp.inf)
    rows = [jnp.max(jnp.where(b == gi, x2, neg), axis=0, keepdims=True)
            for gi in range(G)]
    pooled = jnp.concatenate(rows, axis=0)
    o_ref[...] = (jnp.dot(pooled, wo_ref[...], preferred_element_type=jnp.float32)
                  + bo_ref[...])


@jax.jit
def _final_layer(x, a0, a1, w1t, b1, g, bt, w2t, b2, batch, wot, bout):
    return pl.pallas_call(
        _final_body,
        out_shape=jax.ShapeDtypeStruct((G, OUT), jnp.float32),
    )(x, a0, a1, w1t, b1.reshape(1, D), g.reshape(1, D), bt.reshape(1, D),
      w2t, b2.reshape(1, D), batch.reshape(N, 1), wot, bout.reshape(1, OUT))


def kernel(x, edge_index, egde_weights, batch,
           We0, be0, W1_0, b1_0, g0, bt0, W2_0, b2_0,
           We1, be1, W1_1, b1_1, g1, bt1, W2_1, b2_1,
           Wout, bout):
    pad = E_PAD - E
    srcp = jnp.concatenate(
        [edge_index[0], jnp.zeros((pad,), jnp.int32)]).reshape(NW * CH, C)
    dstp = jnp.concatenate(
        [edge_index[1], jnp.full((pad,), N, jnp.int32)]).reshape(NW * CH, C)
    sd = jnp.stack([srcp, dstp], axis=1).reshape(NW * CH * 2, C)
    ew2 = jnp.concatenate([egde_weights[:, 0], jnp.zeros((pad,), jnp.float32)])
    wp0 = jnp.stack([We0[:, 0], be0])
    wp1 = jnp.stack([We1[:, 0], be1])

    agg0 = _sc_layer(x, sd, ew2, wp0)
    x1 = _mlp_layer(x, agg0[0], agg0[1], W1_0.T, b1_0, g0, bt0, W2_0.T, b2_0)
    agg1 = _sc_layer(x1, sd, ew2, wp1)
    return _final_layer(x1, agg1[0], agg1[1], W1_1.T, b1_1, g1, bt1,
                        W2_1.T, b2_1, batch, Wout.T, bout)


# speedup vs baseline: 1.0140x; 1.0140x over previous
import functools

import jax
import jax.numpy as jnp
from jax import lax
from jax.experimental import pallas as pl
from jax.experimental.pallas import tpu as pltpu
from jax.experimental.pallas import tpu_sc as plsc

N = 10000
E = 320000
D = 128
G = 16
OUT = 5

NC = 2
NS = 16
L = 16
NW = NC * NS

C = 120
NBUF = 3
IB = 4
STEP = 12
CH = -(-(-(-E // (NW * C))) // STEP) * STEP
E_PAD = NW * C * CH
ACC_ROWS = 10112
ZROWS = ACC_ROWS // NS
ZCH = -(-ZROWS // C)
WB = N // NS // 8 * 8
WB_LAST = N - (NS - 1) * WB


def _sc_body(x_hbm, sd_hbm, ew_hbm, w_hbm, out_hbm,
             sd_c, ew_c, rows0, rows1, rows2, w_v,
             gs0, gs1, gs2, ss0, ss1, ss2, is0, is1, is2, is3, acc):
    rows = [rows0, rows1, rows2]
    gsem = [gs0, gs1, gs2]
    ssem = [ss0, ss1, ss2]
    isem = [is0, is1, is2, is3]
    c = lax.axis_index("c")
    s = lax.axis_index("s")
    w = c * NS + s

    pltpu.sync_copy(w_hbm, w_v)

    def zrow(i, _):
        for f in range(D // L):
            rows0[i, pl.ds(f * L, L)] = jnp.zeros((L,), jnp.float32)
        return 0
    lax.fori_loop(0, C, zrow, 0)
    for k in range(ZCH):
        off = min(k * C, ZROWS - C)
        pltpu.sync_copy(rows0, acc.at[pl.ds(s * ZROWS + off, C)])
    plsc.subcore_barrier()

    wecol = [w_v[0, pl.ds(f * L, L)] for f in range(D // L)]
    bev = [w_v[1, pl.ds(f * L, L)] for f in range(D // L)]

    def idx_descs(ci, j):
        g = w * CH + ci
        return (pltpu.make_async_copy(sd_hbm.at[pl.ds(2 * g, 2)],
                                      sd_c.at[pl.ds(2 * j, 2)], isem[j]),
                pltpu.make_async_copy(ew_hbm.at[pl.ds(g * C, C)],
                                      ew_c.at[pl.ds(j * C, C)], isem[j]))

    def idx_start(ci, j):
        for d in idx_descs(ci, j):
            d.start()

    def idx_wait(ci, j):
        for d in idx_descs(ci, j):
            d.wait()

    def gather_desc(j, b):
        return pltpu.make_async_copy(
            x_hbm.at[sd_c.at[2 * j]], rows[b], gsem[b])

    def scatter_desc(j, b):
        return pltpu.make_async_copy(
            rows[b], acc.at[sd_c.at[2 * j + 1]], ssem[b])

    idx_start(0, 0)
    idx_start(1, 1)
    idx_wait(0, 0)
    gather_desc(0, 0).start()

    def outer(k, _):
        for j in range(STEP):
            ci = k * STEP + j
            b = j % NBUF
            b1 = (j + 1) % NBUF
            jj = j % IB

            @pl.when(jnp.logical_and(ci >= 2, ci + 1 < CH))
            def _():
                scatter_desc((jj + 2) % IB, b1).wait()

            @pl.when(ci + 1 < CH)
            def _():
                idx_wait(ci + 1, (jj + 1) % IB)
                gather_desc((jj + 1) % IB, b1).start()

            gather_desc(jj, b).wait()

            @pl.when(ci + 2 < CH)
            def _():
                idx_start(ci + 2, (jj + 2) % IB)

            def edge(i, _):
                ewv = plsc.load_gather(
                    ew_c, [jnp.full((L,), jj * C, jnp.int32) + i])
                for f in range(D // L):
                    v = rows[b][i, pl.ds(f * L, L)]
                    rows[b][i, pl.ds(f * L, L)] = jnp.maximum(
                        v + ewv * wecol[f] + bev[f], 0.0)
                return 0
            lax.fori_loop(0, C, edge, 0)
            scatter_desc(jj, b).start(add=True)
        return 0
    lax.fori_loop(0, CH // STEP, outer, 0)
    for t in range(NBUF):
        ci = CH - NBUF + t
        scatter_desc(ci % IB, ci % NBUF).wait()
    plsc.subcore_barrier()

    @pl.when(s < NS - 1)
    def _():
        pltpu.sync_copy(acc.at[pl.ds(s * WB, WB)],
                        out_hbm.at[c, pl.ds(s * WB, WB)])

    @pl.when(s == NS - 1)
    def _():
        pltpu.sync_copy(acc.at[pl.ds((NS - 1) * WB, WB_LAST)],
                        out_hbm.at[c, pl.ds((NS - 1) * WB, WB_LAST)])


@jax.jit
def _sc_layer(x, sd, ew2, wparams):
    mesh = plsc.VectorSubcoreMesh(core_axis_name="c", subcore_axis_name="s")
    return pl.kernel(
        _sc_body,
        out_type=jax.ShapeDtypeStruct((NC, N, D), jnp.float32),
        mesh=mesh,
        scratch_types=[
            pltpu.VMEM((IB * 2, C), jnp.int32),
            pltpu.VMEM((IB * C,), jnp.float32),
            pltpu.VMEM((C, D), jnp.float32),
            pltpu.VMEM((C, D), jnp.float32),
            pltpu.VMEM((C, D), jnp.float32),
            pltpu.VMEM((2, D), jnp.float32),
        ] + [pltpu.SemaphoreType.DMA] * (NBUF * 2 + IB) + [
            pltpu.VMEM_SHARED((ACC_ROWS, D), jnp.float32),
        ],
        compiler_params=pltpu.CompilerParams(needs_layout_passes=False),
    )(x, sd, ew2, wparams)


def _mlp_body(x_ref, a0_ref, a1_ref, w1_ref, b1_ref, g_ref, bt_ref,
              w2_ref, b2_ref, o_ref):
    h = x_ref[...] + a0_ref[...] + a1_ref[...]
    h1 = jnp.dot(h, w1_ref[...], preferred_element_type=jnp.float32) + b1_ref[...]
    mu = jnp.mean(h1, axis=0, keepdims=True)
    var = jnp.mean((h1 - mu) ** 2, axis=0, keepdims=True)
    hn = (h1 - mu) * lax.rsqrt(var + 1e-5) * g_ref[...] + bt_ref[...]
    r = jnp.maximum(hn, 0.0)
    o_ref[...] = jnp.maximum(
        jnp.dot(r, w2_ref[...], preferred_element_type=jnp.float32) + b2_ref[...],
        0.0)


@jax.jit
def _mlp_layer(x, a0, a1, w1t, b1, g, bt, w2t, b2):
    return pl.pallas_call(
        _mlp_body,
        out_shape=jax.ShapeDtypeStruct((N, D), jnp.float32),
    )(x, a0, a1, w1t, b1.reshape(1, D), g.reshape(1, D), bt.reshape(1, D),
      w2t, b2.reshape(1, D))


def _final_body(x_ref, a0_ref, a1_ref, w1_ref, b1_ref, g_ref, bt_ref,
                w2_ref, b2_ref, batch_ref, wo_ref, bo_ref, o_ref):
    h = x_ref[...] + a0_ref[...] + a1_ref[...]
    h1 = jnp.dot(h, w1_ref[...], preferred_element_type=jnp.float32) + b1_ref[...]
    mu = jnp.mean(h1, axis=0, keepdims=True)
    var = jnp.mean((h1 - mu) ** 2, axis=0, keepdims=True)
    hn = (h1 - mu) * lax.rsqrt(var + 1e-5) * g_ref[...] + bt_ref[...]
    r = jnp.maximum(hn, 0.0)
    x2 = jnp.maximum(
        jnp.dot(r, w2_ref[...], preferred_element_type=jnp.float32) + b2_ref[...],
        0.0)
    b = batch_ref[...]
    neg = jnp.float32(-jnp.inf)
    rows = [jnp.max(jnp.where(b == gi, x2, neg), axis=0, keepdims=True)
            for gi in range(G)]
    pooled = jnp.concatenate(rows, axis=0)
    o_ref[...] = (jnp.dot(pooled, wo_ref[...], preferred_element_type=jnp.float32)
                  + bo_ref[...])


@jax.jit
def _final_layer(x, a0, a1, w1t, b1, g, bt, w2t, b2, batch, wot, bout):
    return pl.pallas_call(
        _final_body,
        out_shape=jax.ShapeDtypeStruct((G, OUT), jnp.float32),
    )(x, a0, a1, w1t, b1.reshape(1, D), g.reshape(1, D), bt.reshape(1, D),
      w2t, b2.reshape(1, D), batch.reshape(N, 1), wot, bout.reshape(1, OUT))


def kernel(x, edge_index, egde_weights, batch,
           We0, be0, W1_0, b1_0, g0, bt0, W2_0, b2_0,
           We1, be1, W1_1, b1_1, g1, bt1, W2_1, b2_1,
           Wout, bout):
    pad = E_PAD - E
    srcp = jnp.concatenate(
        [edge_index[0], jnp.zeros((pad,), jnp.int32)]).reshape(NW * CH, C)
    dstp = jnp.concatenate(
        [edge_index[1], jnp.full((pad,), N, jnp.int32)]).reshape(NW * CH, C)
    sd = jnp.stack([srcp, dstp], axis=1).reshape(NW * CH * 2, C)
    ew2 = jnp.concatenate([egde_weights[:, 0], jnp.zeros((pad,), jnp.float32)])
    wp0 = jnp.stack([We0[:, 0], be0])
    wp1 = jnp.stack([We1[:, 0], be1])

    agg0 = _sc_layer(x, sd, ew2, wp0)
    x1 = _mlp_layer(x, agg0[0], agg0[1], W1_0.T, b1_0, g0, bt0, W2_0.T, b2_0)
    agg1 = _sc_layer(x1, sd, ew2, wp1)
    return _final_layer(x1, agg1[0], agg1[1], W1_1.T, b1_1, g1, bt1,
                        W2_1.T, b2_1, batch, Wout.T, bout)
